# baseline (speedup 1.0000x reference)
import functools

import jax
import jax.numpy as jnp
from jax import lax
from jax.experimental import pallas as pl
from jax.experimental.pallas import tpu as pltpu
from jax.experimental.pallas import tpu_sc as plsc

N = 10000
E = 160000
D_IN = 256
D_H = 256
HALF = 128

NC = 2
NS = 16
L = 16

RPT = 640
RPT_LAST = N - RPT * (NS - 1)
ROWS_PAD = RPT * NS

CB = 128
NCB = E // CB
CH = 64

RB = 1000
NRB = N // RB

CPW1 = NCB // (NC * NS)
PAIRS1 = (CPW1 - 2) // 2

BPT3 = NCB // NS
CPT3 = 2 * BPT3
TRIADS3 = CPT3 // 3


def _mesh():
    return plsc.VectorSubcoreMesh(core_axis_name="c", subcore_axis_name="s",
                                  num_cores=NC, num_subcores=NS)


def _row_slab(s):
    return s * RPT, s == NS - 1


def _deg_count(ev):
    @functools.partial(
        pl.kernel,
        out_type=jax.ShapeDtypeStruct((NC * N,), jnp.float32),
        mesh=_mesh(),
        scratch_types=[
            pltpu.VMEM_SHARED((ROWS_PAD,), jnp.float32),
            pltpu.VMEM((RPT,), jnp.float32),
            pltpu.VMEM((CB,), jnp.float32),
            pltpu.VMEM((CPW1, 1, CB), jnp.int32),
            pltpu.VMEM((CB,), jnp.int32),
            pltpu.VMEM((CB,), jnp.int32),
            pltpu.SemaphoreType.DMA,
            pltpu.SemaphoreType.DMA,
        ],
    )
    def k(ev_hbm, deg2_hbm, deg, zerov, onesb, dstall, dstb0, dstb1,
          sems0, sems1):
        c = lax.axis_index("c")
        s = lax.axis_index("s")
        base_r, is_last = _row_slab(s)
        w = c * NS + s
        first = w * CPW1

        pltpu.sync_copy(ev_hbm.at[1, pl.ds(first, CPW1)], dstall)
        for g in range(CB // L):
            onesb[pl.ds(g * L, L)] = jnp.full((L,), 1.0, jnp.float32)
        for g in range(RPT // L):
            zerov[pl.ds(g * L, L)] = jnp.zeros((L,), jnp.float32)
        pltpu.sync_copy(zerov, deg.at[pl.ds(base_r, RPT)])
        plsc.subcore_barrier()

        def move(i, dstb):
            for g in range(CB // L):
                dstb[pl.ds(g * L, L)] = dstall[i, 0, pl.ds(g * L, L)]

        def fire(dstb, sem):
            pltpu.async_copy(onesb, deg.at[dstb], sem, add=True)

        def drain(dstb, sem):
            pltpu.make_async_copy(onesb, deg.at[dstb], sem).wait()

        move(0, dstb0)
        fire(dstb0, sems0)

        def pair(p, carry):
            i0 = 2 * p + 1
            move(i0, dstb1)
            fire(dstb1, sems1)
            drain(dstb0, sems0)
            move(i0 + 1, dstb0)
            fire(dstb0, sems0)
            drain(dstb1, sems1)
            return carry

        lax.fori_loop(0, PAIRS1, pair, 0)
        drain(dstb0, sems0)
        for i in (CPW1 - 2, CPW1 - 1):
            move(i, dstb0)
            pltpu.sync_copy(onesb, deg.at[dstb0], add=True)

        @pl.when(w < 2)
        def _():
            pltpu.sync_copy(ev_hbm.at[1, NC * NS * CPW1 + w, 0], dstb0)
            pltpu.sync_copy(onesb, deg.at[dstb0], add=True)

        plsc.subcore_barrier()

        pltpu.sync_copy(deg.at[pl.ds(base_r, RPT)], zerov)

        @pl.when(jnp.logical_not(is_last))
        def _():
            pltpu.sync_copy(zerov,
                            deg2_hbm.at[pl.ds(c * N + base_r, RPT)])

        @pl.when(is_last)
        def _():
            pltpu.sync_copy(zerov.at[pl.ds(0, RPT_LAST)],
                            deg2_hbm.at[pl.ds(c * N + base_r, RPT_LAST)])

    return k(ev)


def _mm_body(x_ref, w_ref, deg_ref, o_ref):
    i = pl.program_id(0)
    d = deg_ref[0, i, :] + deg_ref[1, i, :] + 1.0
    dis = lax.rsqrt(d)
    h = jnp.dot(x_ref[...], w_ref[...], preferred_element_type=jnp.float32)
    h = h * dis[:, None]
    o_ref[0, :, :] = h[:, :HALF]
    o_ref[1, :, :] = h[:, HALF:]


def _matmul_scaled(x, W, degt):
    return pl.pallas_call(
        _mm_body,
        grid=(NRB,),
        in_specs=[
            pl.BlockSpec((RB, D_IN), lambda i: (i, 0)),
            pl.BlockSpec((D_IN, D_H), lambda i: (0, 0)),
            pl.BlockSpec((NC, NRB, RB), lambda i: (0, 0, 0)),
        ],
        out_specs=pl.BlockSpec((NC, RB, HALF), lambda i: (0, i, 0)),
        out_shape=jax.ShapeDtypeStruct((2, N, HALF), jnp.float32),
    )(x, W, degt)


def _edge_accumulate(hs2, ev):
    @functools.partial(
        pl.kernel,
        out_type=jax.ShapeDtypeStruct((NC, N, HALF), jnp.float32),
        mesh=_mesh(),
        scratch_types=[
            pltpu.VMEM_SHARED((N, HALF), jnp.float32),
            pltpu.VMEM((CH, HALF), jnp.float32),
            pltpu.VMEM((CH, HALF), jnp.float32),
            pltpu.VMEM((CH, HALF), jnp.float32),
            pltpu.VMEM((BPT3, 1, CB), jnp.int32),
            pltpu.VMEM((BPT3, 1, CB), jnp.int32),
            pltpu.VMEM((CH,), jnp.int32),
            pltpu.VMEM((CH,), jnp.int32),
            pltpu.VMEM((CH,), jnp.int32),
            pltpu.VMEM((CH,), jnp.int32),
            pltpu.VMEM((CH,), jnp.int32),
            pltpu.VMEM((CH,), jnp.int32),
            pltpu.VMEM((CB,), jnp.int32),
            pltpu.VMEM((CB,), jnp.int32),
            pltpu.SemaphoreType.DMA,
            pltpu.SemaphoreType.DMA,
            pltpu.SemaphoreType.DMA,
            pltpu.SemaphoreType.DMA,
            pltpu.SemaphoreType.DMA,
            pltpu.SemaphoreType.DMA,
        ],
    )
    def k(hs_hbm, ev_hbm, acc2_hbm, acc,
          rows0, rows1, rows2, srcall, dstall,
          srcb0, srcb1, srcb2, dstb0, dstb1, dstb2, srct, dstt,
          semg0, semg1, semg2, sems0, sems1, sems2):
        rows = [rows0, rows1, rows2]
        srcb = [srcb0, srcb1, srcb2]
        dstb = [dstb0, dstb1, dstb2]
        semg = [semg0, semg1, semg2]
        sems = [sems0, sems1, sems2]

        c = lax.axis_index("c")
        s = lax.axis_index("s")
        base_r, is_last = _row_slab(s)
        bfirst = s * BPT3

        pltpu.sync_copy(ev_hbm.at[0, pl.ds(bfirst, BPT3)], srcall)
        pltpu.sync_copy(ev_hbm.at[1, pl.ds(bfirst, BPT3)], dstall)

        @pl.when(jnp.logical_not(is_last))
        def _():
            pltpu.sync_copy(hs_hbm.at[c, pl.ds(base_r, RPT)],
                            acc.at[pl.ds(base_r, RPT)])

        @pl.when(is_last)
        def _():
            pltpu.sync_copy(hs_hbm.at[c, pl.ds(base_r, RPT_LAST)],
                            acc.at[pl.ds(base_r, RPT_LAST)])

        plsc.subcore_barrier()

        def launch(i, a):
            off = (i % 2) * CH
            for g in range(CH // L):
                srcb[a][pl.ds(g * L, L)] = srcall[i // 2, 0,
                                                  pl.ds(off + g * L, L)]
                dstb[a][pl.ds(g * L, L)] = dstall[i // 2, 0,
                                                  pl.ds(off + g * L, L)]
            pltpu.async_copy(hs_hbm.at[c].at[srcb[a]], rows[a], semg[a])

        def retire(a):
            pltpu.make_async_copy(hs_hbm.at[c].at[srcb[a]], rows[a],
                                  semg[a]).wait()
            pltpu.async_copy(rows[a], acc.at[dstb[a]], sems[a], add=True)

        def wait_scat(a):
            pltpu.make_async_copy(rows[a], acc.at[dstb[a]], sems[a]).wait()

        launch(0, 0)

        def triad(q, carry):
            for a in range(3):
                i = 3 * q + a
                retire(a)
                nxt = (a + 1) % 3

                if a == 2:
                    @pl.when(q < TRIADS3 - 1)
                    def _():
                        wait_scat(nxt)
                        launch(i + 1, nxt)
                else:
                    @pl.when(q > 0)
                    def _():
                        wait_scat(nxt)
                    launch(i + 1, nxt)
            return carry

        lax.fori_loop(0, TRIADS3, triad, 0)
        wait_scat(0)
        wait_scat(1)
        wait_scat(2)

        @pl.when(s < 4)
        def _():
            blk = NS * BPT3 + s // 2
            off = (s % 2) * CH
            pltpu.sync_copy(ev_hbm.at[0, blk, 0], srct)
            pltpu.sync_copy(ev_hbm.at[1, blk, 0], dstt)
            for g in range(CH // L):
                srcb[0][pl.ds(g * L, L)] = srct[pl.ds(off + g * L, L)]
                dstb[0][pl.ds(g * L, L)] = dstt[pl.ds(off + g * L, L)]
            pltpu.async_copy(hs_hbm.at[c].at[srcb[0]], rows0, semg0).wait()
            pltpu.sync_copy(rows0, acc.at[dstb[0]], add=True)

        plsc.subcore_barrier()

        @pl.when(jnp.logical_not(is_last))
        def _():
            pltpu.sync_copy(acc.at[pl.ds(base_r, RPT)],
                            acc2_hbm.at[c, pl.ds(base_r, RPT)])

        @pl.when(is_last)
        def _():
            pltpu.sync_copy(acc.at[pl.ds(base_r, RPT_LAST)],
                            acc2_hbm.at[c, pl.ds(base_r, RPT_LAST)])

    return k(hs2, ev)


def _ro_body(acc_ref, deg_ref, b_ref, o_ref):
    i = pl.program_id(0)
    d = deg_ref[0, i, :] + deg_ref[1, i, :] + 1.0
    dis = lax.rsqrt(d)
    a = jnp.concatenate([acc_ref[0], acc_ref[1]], axis=1)
    o_ref[...] = jnp.maximum(a * dis[:, None] + b_ref[0, :][None, :], 0.0)


def _readout(acc2, degt, b):
    return pl.pallas_call(
        _ro_body,
        grid=(NRB,),
        in_specs=[
            pl.BlockSpec((NC, RB, HALF), lambda i: (0, i, 0)),
            pl.BlockSpec((NC, NRB, RB), lambda i: (0, 0, 0)),
            pl.BlockSpec((1, D_H), lambda i: (0, 0)),
        ],
        out_specs=pl.BlockSpec((RB, D_H), lambda i: (i, 0)),
        out_shape=jax.ShapeDtypeStruct((N, D_H), jnp.float32),
    )(acc2, degt, b.reshape(1, D_H))


def kernel(x, edge_index, W, b):
    ev = edge_index.reshape(2, NCB, 1, CB)
    deg2 = _deg_count(ev)
    degt = deg2.reshape(NC, NRB, RB)
    hs2 = _matmul_scaled(x, W, degt)
    acc2 = _edge_accumulate(hs2, ev)
    return _readout(acc2, degt, b)


# speedup vs baseline: 1.3179x; 1.3179x over previous
import functools

import jax
import jax.numpy as jnp
from jax import lax
from jax.experimental import pallas as pl
from jax.experimental.pallas import tpu as pltpu
from jax.experimental.pallas import tpu_sc as plsc

N = 10000
E = 160000
D_IN = 256
D_H = 256
HALF = 128

NC = 2
NS = 16
L = 16

RPT = 640
RPT_LAST = N - RPT * (NS - 1)
ROWS_PAD = RPT * NS

CH = 64
NCH = E // CH

RB = 1000
NRB = N // RB

CPW1 = NCH // (NC * NS)
PAIRS1 = (CPW1 - 2) // 2

CPT3 = NCH // NS
QUADS3 = CPT3 // 4


def _mesh():
    return plsc.VectorSubcoreMesh(core_axis_name="c", subcore_axis_name="s",
                                  num_cores=NC, num_subcores=NS)


def _row_slab(s):
    return s * RPT, s == NS - 1


def _deg_count(ev):
    @functools.partial(
        pl.kernel,
        out_type=jax.ShapeDtypeStruct((NC * N,), jnp.float32),
        mesh=_mesh(),
        scratch_types=[
            pltpu.VMEM_SHARED((ROWS_PAD,), jnp.float32),
            pltpu.VMEM((RPT,), jnp.float32),
            pltpu.VMEM((CH,), jnp.float32),
            pltpu.VMEM((CPW1, 1, CH), jnp.int32),
            pltpu.VMEM((CH,), jnp.int32),
            pltpu.VMEM((CH,), jnp.int32),
            pltpu.SemaphoreType.DMA,
            pltpu.SemaphoreType.DMA,
        ],
    )
    def k(ev_hbm, deg2_hbm, deg, zerov, onesb, dstall, dstb0, dstb1,
          sems0, sems1):
        c = lax.axis_index("c")
        s = lax.axis_index("s")
        base_r, is_last = _row_slab(s)
        w = c * NS + s
        first = w * CPW1

        pltpu.sync_copy(ev_hbm.at[1, pl.ds(first, CPW1)], dstall)
        for g in range(CH // L):
            onesb[pl.ds(g * L, L)] = jnp.full((L,), 1.0, jnp.float32)
        for g in range(RPT // L):
            zerov[pl.ds(g * L, L)] = jnp.zeros((L,), jnp.float32)
        pltpu.sync_copy(zerov, deg.at[pl.ds(base_r, RPT)])
        plsc.subcore_barrier()

        def move(i, dstb):
            for g in range(CH // L):
                dstb[pl.ds(g * L, L)] = dstall[i, 0, pl.ds(g * L, L)]

        def fire(dstb, sem):
            pltpu.async_copy(onesb, deg.at[dstb], sem, add=True)

        def drain(dstb, sem):
            pltpu.make_async_copy(onesb, deg.at[dstb], sem).wait()

        move(0, dstb0)
        fire(dstb0, sems0)

        def pair(p, carry):
            i0 = 2 * p + 1
            move(i0, dstb1)
            fire(dstb1, sems1)
            drain(dstb0, sems0)
            move(i0 + 1, dstb0)
            fire(dstb0, sems0)
            drain(dstb1, sems1)
            return carry

        lax.fori_loop(0, PAIRS1, pair, 0)
        drain(dstb0, sems0)
        move(CPW1 - 1, dstb0)
        pltpu.sync_copy(onesb, deg.at[dstb0], add=True)

        @pl.when(w < 4)
        def _():
            pltpu.sync_copy(ev_hbm.at[1, NC * NS * CPW1 + w, 0], dstb0)
            pltpu.sync_copy(onesb, deg.at[dstb0], add=True)

        plsc.subcore_barrier()

        pltpu.sync_copy(deg.at[pl.ds(base_r, RPT)], zerov)

        @pl.when(jnp.logical_not(is_last))
        def _():
            pltpu.sync_copy(zerov,
                            deg2_hbm.at[pl.ds(c * N + base_r, RPT)])

        @pl.when(is_last)
        def _():
            pltpu.sync_copy(zerov.at[pl.ds(0, RPT_LAST)],
                            deg2_hbm.at[pl.ds(c * N + base_r, RPT_LAST)])

    return k(ev)


def _mm_body(x_ref, w_ref, deg_ref, o_ref):
    i = pl.program_id(0)
    d = deg_ref[0, i, :] + deg_ref[1, i, :] + 1.0
    dis = lax.rsqrt(d)
    h = jnp.dot(x_ref[...], w_ref[...], preferred_element_type=jnp.float32)
    h = h * dis[:, None]
    o_ref[0, :, :] = h[:, :HALF]
    o_ref[1, :, :] = h[:, HALF:]


def _matmul_scaled(x, W, degt):
    return pl.pallas_call(
        _mm_body,
        grid=(NRB,),
        in_specs=[
            pl.BlockSpec((RB, D_IN), lambda i: (i, 0)),
            pl.BlockSpec((D_IN, D_H), lambda i: (0, 0)),
            pl.BlockSpec((NC, NRB, RB), lambda i: (0, 0, 0)),
        ],
        out_specs=pl.BlockSpec((NC, RB, HALF), lambda i: (0, i, 0)),
        out_shape=jax.ShapeDtypeStruct((2, N, HALF), jnp.float32),
    )(x, W, degt)


def _edge_accumulate(hs2, ev, ev128):
    @functools.partial(
        pl.kernel,
        out_type=jax.ShapeDtypeStruct((NC, N, HALF), jnp.float32),
        mesh=_mesh(),
        scratch_types=[
            pltpu.VMEM_SHARED((N, HALF), jnp.float32),
            pltpu.VMEM((CH, HALF), jnp.float32),
            pltpu.VMEM((CH, HALF), jnp.float32),
            pltpu.VMEM((CH, HALF), jnp.float32),
            pltpu.VMEM((CH, HALF), jnp.float32),
            pltpu.VMEM((CPT3 // 2, 1, 2 * CH), jnp.int32),
            pltpu.VMEM((CH,), jnp.int32),
            pltpu.VMEM((CH,), jnp.int32),
            pltpu.VMEM((CH,), jnp.int32),
            pltpu.VMEM((CH,), jnp.int32),
            pltpu.VMEM((CH,), jnp.int32),
            pltpu.VMEM((CH,), jnp.int32),
            pltpu.VMEM((CH,), jnp.int32),
            pltpu.VMEM((CH,), jnp.int32),
            pltpu.SemaphoreType.DMA,
            pltpu.SemaphoreType.DMA,
            pltpu.SemaphoreType.DMA,
            pltpu.SemaphoreType.DMA,
            pltpu.SemaphoreType.DMA,
            pltpu.SemaphoreType.DMA,
            pltpu.SemaphoreType.DMA,
            pltpu.SemaphoreType.DMA,
            pltpu.SemaphoreType.DMA,
            pltpu.SemaphoreType.DMA,
            pltpu.SemaphoreType.DMA,
            pltpu.SemaphoreType.DMA,
        ],
    )
    def k(hs_hbm, ev_hbm, ev128_hbm, acc2_hbm, acc,
          rows0, rows1, rows2, rows3, srcall,
          srcb0, srcb1, srcb2, srcb3, dstb0, dstb1, dstb2, dstb3,
          semg0, semg1, semg2, semg3, semi0, semi1, semi2, semi3,
          sems0, sems1, sems2, sems3):
        rows = [rows0, rows1, rows2, rows3]
        srcb = [srcb0, srcb1, srcb2, srcb3]
        dstb = [dstb0, dstb1, dstb2, dstb3]
        semg = [semg0, semg1, semg2, semg3]
        semi = [semi0, semi1, semi2, semi3]
        sems = [sems0, sems1, sems2, sems3]

        c = lax.axis_index("c")
        s = lax.axis_index("s")
        base_r, is_last = _row_slab(s)
        first = s * CPT3

        pltpu.sync_copy(ev128_hbm.at[0, pl.ds(s * (CPT3 // 2), CPT3 // 2)],
                        srcall)

        @pl.when(jnp.logical_not(is_last))
        def _():
            pltpu.sync_copy(hs_hbm.at[c, pl.ds(base_r, RPT)],
                            acc.at[pl.ds(base_r, RPT)])

        @pl.when(is_last)
        def _():
            pltpu.sync_copy(hs_hbm.at[c, pl.ds(base_r, RPT_LAST)],
                            acc.at[pl.ds(base_r, RPT_LAST)])

        plsc.subcore_barrier()

        def launch(i, a):
            idx = srcall.at[i // 2, 0, pl.ds((i % 2) * CH, CH)]
            pltpu.async_copy(hs_hbm.at[c].at[idx], rows[a], semg[a])
            pltpu.async_copy(ev_hbm.at[1, first + i, 0], dstb[a], semi[a])

        def retire(a):
            pltpu.make_async_copy(hs_hbm.at[c].at[srcb[a]], rows[a],
                                  semg[a]).wait()
            pltpu.make_async_copy(ev_hbm.at[1, 0, 0], dstb[a], semi[a]).wait()
            pltpu.async_copy(rows[a], acc.at[dstb[a]], sems[a], add=True)

        def wait_scat(a):
            pltpu.make_async_copy(rows[a], acc.at[dstb[a]], sems[a]).wait()

        launch(0, 0)
        launch(1, 1)

        def quad(q, carry):
            for a in range(4):
                i = 4 * q + a
                retire(a)
                nxt = a + 2 if a < 2 else a - 2

                if a < 2:
                    @pl.when(q > 0)
                    def _():
                        wait_scat(nxt)
                    launch(i + 2, nxt)
                else:
                    @pl.when(q < QUADS3 - 1)
                    def _():
                        wait_scat(nxt)
                        launch(i + 2, nxt)
            return carry

        lax.fori_loop(0, QUADS3, quad, 0)
        wait_scat(0)
        wait_scat(1)
        wait_scat(2)
        wait_scat(3)

        @pl.when(s < 4)
        def _():
            pltpu.sync_copy(ev_hbm.at[0, NS * CPT3 + s, 0], srcb0)
            pltpu.sync_copy(ev_hbm.at[1, NS * CPT3 + s, 0], dstb0)
            pltpu.async_copy(hs_hbm.at[c].at[srcb0], rows0, semg0).wait()
            pltpu.sync_copy(rows0, acc.at[dstb0], add=True)

        plsc.subcore_barrier()

        @pl.when(jnp.logical_not(is_last))
        def _():
            pltpu.sync_copy(acc.at[pl.ds(base_r, RPT)],
                            acc2_hbm.at[c, pl.ds(base_r, RPT)])

        @pl.when(is_last)
        def _():
            pltpu.sync_copy(acc.at[pl.ds(base_r, RPT_LAST)],
                            acc2_hbm.at[c, pl.ds(base_r, RPT_LAST)])

    return k(hs2, ev, ev128)


def _ro_body(acc_ref, deg_ref, b_ref, o_ref):
    i = pl.program_id(0)
    d = deg_ref[0, i, :] + deg_ref[1, i, :] + 1.0
    dis = lax.rsqrt(d)
    a = jnp.concatenate([acc_ref[0], acc_ref[1]], axis=1)
    o_ref[...] = jnp.maximum(a * dis[:, None] + b_ref[0, :][None, :], 0.0)


def _readout(acc2, degt, b):
    return pl.pallas_call(
        _ro_body,
        grid=(NRB,),
        in_specs=[
            pl.BlockSpec((NC, RB, HALF), lambda i: (0, i, 0)),
            pl.BlockSpec((NC, NRB, RB), lambda i: (0, 0, 0)),
            pl.BlockSpec((1, D_H), lambda i: (0, 0)),
        ],
        out_specs=pl.BlockSpec((RB, D_H), lambda i: (i, 0)),
        out_shape=jax.ShapeDtypeStruct((N, D_H), jnp.float32),
    )(acc2, degt, b.reshape(1, D_H))


def kernel(x, edge_index, W, b):
    ev = edge_index.reshape(2, NCH, 1, CH)
    ev128 = edge_index.reshape(2, NCH // 2, 1, 2 * CH)
    deg2 = _deg_count(ev)
    degt = deg2.reshape(NC, NRB, RB)
    hs2 = _matmul_scaled(x, W, degt)
    acc2 = _edge_accumulate(hs2, ev, ev128)
    return _readout(acc2, degt, b)


# speedup vs baseline: 1.4557x; 1.1045x over previous
import functools

import jax
import jax.numpy as jnp
from jax import lax
from jax.experimental import pallas as pl
from jax.experimental.pallas import tpu as pltpu
from jax.experimental.pallas import tpu_sc as plsc

N = 10000
E = 160000
D_IN = 256
D_H = 256
HALF = 128

NC = 2
NS = 16
L = 16

RPT = 640
RPT_LAST = N - RPT * (NS - 1)
ROWS_PAD = RPT * NS

CH = 64
NCH = E // CH

RB = 1000
NRB = N // RB

CPW1 = NCH // (NC * NS)
PAIRS1 = (CPW1 - 2) // 2

CPT3 = NCH // NS
QUADS3 = CPT3 // 4


def _mesh():
    return plsc.VectorSubcoreMesh(core_axis_name="c", subcore_axis_name="s",
                                  num_cores=NC, num_subcores=NS)


def _row_slab(s):
    return s * RPT, s == NS - 1


def _deg_count(ev):
    @functools.partial(
        pl.kernel,
        out_type=jax.ShapeDtypeStruct((NC * N,), jnp.float32),
        mesh=_mesh(),
        scratch_types=[
            pltpu.VMEM_SHARED((ROWS_PAD,), jnp.float32),
            pltpu.VMEM((RPT,), jnp.float32),
            pltpu.VMEM((CH,), jnp.float32),
            pltpu.VMEM((CPW1, 1, CH), jnp.int32),
            pltpu.VMEM((CH,), jnp.int32),
            pltpu.VMEM((CH,), jnp.int32),
            pltpu.SemaphoreType.DMA,
            pltpu.SemaphoreType.DMA,
        ],
    )
    def k(ev_hbm, deg2_hbm, deg, zerov, onesb, dstall, dstb0, dstb1,
          sems0, sems1):
        c = lax.axis_index("c")
        s = lax.axis_index("s")
        base_r, is_last = _row_slab(s)
        w = c * NS + s
        first = w * CPW1

        pltpu.sync_copy(ev_hbm.at[1, pl.ds(first, CPW1)], dstall)
        for g in range(CH // L):
            onesb[pl.ds(g * L, L)] = jnp.full((L,), 1.0, jnp.float32)
        for g in range(RPT // L):
            zerov[pl.ds(g * L, L)] = jnp.zeros((L,), jnp.float32)
        pltpu.sync_copy(zerov, deg.at[pl.ds(base_r, RPT)])
        plsc.subcore_barrier()

        def move(i, dstb):
            for g in range(CH // L):
                dstb[pl.ds(g * L, L)] = dstall[i, 0, pl.ds(g * L, L)]

        def fire(dstb, sem):
            pltpu.async_copy(onesb, deg.at[dstb], sem, add=True)

        def drain(dstb, sem):
            pltpu.make_async_copy(onesb, deg.at[dstb], sem).wait()

        move(0, dstb0)
        fire(dstb0, sems0)

        def pair(p, carry):
            i0 = 2 * p + 1
            move(i0, dstb1)
            fire(dstb1, sems1)
            drain(dstb0, sems0)
            move(i0 + 1, dstb0)
            fire(dstb0, sems0)
            drain(dstb1, sems1)
            return carry

        lax.fori_loop(0, PAIRS1, pair, 0)
        drain(dstb0, sems0)
        move(CPW1 - 1, dstb0)
        pltpu.sync_copy(onesb, deg.at[dstb0], add=True)

        @pl.when(w < 4)
        def _():
            pltpu.sync_copy(ev_hbm.at[1, NC * NS * CPW1 + w, 0], dstb0)
            pltpu.sync_copy(onesb, deg.at[dstb0], add=True)

        plsc.subcore_barrier()

        pltpu.sync_copy(deg.at[pl.ds(base_r, RPT)], zerov)

        @pl.when(jnp.logical_not(is_last))
        def _():
            pltpu.sync_copy(zerov,
                            deg2_hbm.at[pl.ds(c * N + base_r, RPT)])

        @pl.when(is_last)
        def _():
            pltpu.sync_copy(zerov.at[pl.ds(0, RPT_LAST)],
                            deg2_hbm.at[pl.ds(c * N + base_r, RPT_LAST)])

    return k(ev)


def _mm_body(x_ref, w_ref, deg_ref, o_ref):
    i = pl.program_id(0)
    d = deg_ref[0, i, :] + deg_ref[1, i, :] + 1.0
    dis = lax.rsqrt(d)
    h = jnp.dot(x_ref[...], w_ref[...], preferred_element_type=jnp.float32)
    h = h * dis[:, None]
    o_ref[0, :, :] = h[:, :HALF]
    o_ref[1, :, :] = h[:, HALF:]


def _matmul_scaled(x, W, degt):
    return pl.pallas_call(
        _mm_body,
        grid=(NRB,),
        in_specs=[
            pl.BlockSpec((RB, D_IN), lambda i: (i, 0)),
            pl.BlockSpec((D_IN, D_H), lambda i: (0, 0)),
            pl.BlockSpec((NC, NRB, RB), lambda i: (0, 0, 0)),
        ],
        out_specs=pl.BlockSpec((NC, RB, HALF), lambda i: (0, i, 0)),
        out_shape=jax.ShapeDtypeStruct((2, N, HALF), jnp.float32),
    )(x, W, degt)


def _edge_accumulate(hs2, ev, ev128):
    @functools.partial(
        pl.kernel,
        out_type=jax.ShapeDtypeStruct((NC, N, HALF), jnp.float32),
        mesh=_mesh(),
        scratch_types=[
            pltpu.VMEM_SHARED((N, HALF), jnp.float32),
            pltpu.VMEM((CH, HALF), jnp.float32),
            pltpu.VMEM((CH, HALF), jnp.float32),
            pltpu.VMEM((CH, HALF), jnp.float32),
            pltpu.VMEM((CH, HALF), jnp.float32),
            pltpu.VMEM((CPT3 // 2, 1, 2 * CH), jnp.int32),
            pltpu.VMEM((CH,), jnp.int32),
            pltpu.VMEM((CH,), jnp.int32),
            pltpu.VMEM((CH,), jnp.int32),
            pltpu.VMEM((CH,), jnp.int32),
            pltpu.VMEM((CH,), jnp.int32),
            pltpu.VMEM((CH,), jnp.int32),
            pltpu.VMEM((CH,), jnp.int32),
            pltpu.VMEM((CH,), jnp.int32),
            pltpu.SemaphoreType.DMA,
            pltpu.SemaphoreType.DMA,
            pltpu.SemaphoreType.DMA,
            pltpu.SemaphoreType.DMA,
            pltpu.SemaphoreType.DMA,
            pltpu.SemaphoreType.DMA,
            pltpu.SemaphoreType.DMA,
            pltpu.SemaphoreType.DMA,
            pltpu.SemaphoreType.DMA,
            pltpu.SemaphoreType.DMA,
            pltpu.SemaphoreType.DMA,
            pltpu.SemaphoreType.DMA,
        ],
    )
    def k(hs_hbm, ev_hbm, ev128_hbm, acc2_hbm, acc,
          rows0, rows1, rows2, rows3, srcall,
          srcb0, srcb1, srcb2, srcb3, dstb0, dstb1, dstb2, dstb3,
          semg0, semg1, semg2, semg3, semi0, semi1, semi2, semi3,
          sems0, sems1, sems2, sems3):
        rows = [rows0, rows1, rows2, rows3]
        srcb = [srcb0, srcb1, srcb2, srcb3]
        dstb = [dstb0, dstb1, dstb2, dstb3]
        semg = [semg0, semg1, semg2, semg3]
        semi = [semi0, semi1, semi2, semi3]
        sems = [sems0, sems1, sems2, sems3]

        c = lax.axis_index("c")
        s = lax.axis_index("s")
        base_r, is_last = _row_slab(s)
        first = s * CPT3

        pltpu.sync_copy(ev128_hbm.at[0, pl.ds(s * (CPT3 // 2), CPT3 // 2)],
                        srcall)

        @pl.when(jnp.logical_not(is_last))
        def _():
            pltpu.sync_copy(hs_hbm.at[c, pl.ds(base_r, RPT)],
                            acc.at[pl.ds(base_r, RPT)])

        @pl.when(is_last)
        def _():
            pltpu.sync_copy(hs_hbm.at[c, pl.ds(base_r, RPT_LAST)],
                            acc.at[pl.ds(base_r, RPT_LAST)])

        plsc.subcore_barrier()

        def launch(i, a):
            idx = srcall.at[i // 2, 0, pl.ds((i % 2) * CH, CH)]
            pltpu.async_copy(hs_hbm.at[c].at[idx], rows[a], semg[a])
            pltpu.async_copy(ev_hbm.at[1, first + i, 0], dstb[a], semi[a])

        def retire(a):
            pltpu.make_async_copy(hs_hbm.at[c].at[srcb[a]], rows[a],
                                  semg[a]).wait()
            pltpu.make_async_copy(ev_hbm.at[1, 0, 0], dstb[a], semi[a]).wait()
            pltpu.async_copy(rows[a], acc.at[dstb[a]], sems[a], add=True)

        def wait_scat(a):
            pltpu.make_async_copy(rows[a], acc.at[dstb[a]], sems[a]).wait()

        launch(0, 0)
        launch(1, 1)
        launch(2, 2)

        def quad(q, carry):
            for a in range(4):
                i = 4 * q + a
                retire(a)
                nxt = (a + 3) % 4

                if a == 0:
                    @pl.when(q > 0)
                    def _():
                        wait_scat(nxt)
                    launch(i + 3, nxt)
                else:
                    wait_scat(nxt)

                    @pl.when(q < QUADS3 - 1)
                    def _():
                        launch(i + 3, nxt)
            return carry

        lax.fori_loop(0, QUADS3, quad, 0)
        wait_scat(3)

        @pl.when(s < 4)
        def _():
            pltpu.sync_copy(ev_hbm.at[0, NS * CPT3 + s, 0], srcb0)
            pltpu.sync_copy(ev_hbm.at[1, NS * CPT3 + s, 0], dstb0)
            pltpu.async_copy(hs_hbm.at[c].at[srcb0], rows0, semg0).wait()
            pltpu.sync_copy(rows0, acc.at[dstb0], add=True)

        plsc.subcore_barrier()

        @pl.when(jnp.logical_not(is_last))
        def _():
            pltpu.sync_copy(acc.at[pl.ds(base_r, RPT)],
                            acc2_hbm.at[c, pl.ds(base_r, RPT)])

        @pl.when(is_last)
        def _():
            pltpu.sync_copy(acc.at[pl.ds(base_r, RPT_LAST)],
                            acc2_hbm.at[c, pl.ds(base_r, RPT_LAST)])

    return k(hs2, ev, ev128)


def _ro_body(acc_ref, deg_ref, b_ref, o_ref):
    i = pl.program_id(0)
    d = deg_ref[0, i, :] + deg_ref[1, i, :] + 1.0
    dis = lax.rsqrt(d)
    a = jnp.concatenate([acc_ref[0], acc_ref[1]], axis=1)
    o_ref[...] = jnp.maximum(a * dis[:, None] + b_ref[0, :][None, :], 0.0)


def _readout(acc2, degt, b):
    return pl.pallas_call(
        _ro_body,
        grid=(NRB,),
        in_specs=[
            pl.BlockSpec((NC, RB, HALF), lambda i: (0, i, 0)),
            pl.BlockSpec((NC, NRB, RB), lambda i: (0, 0, 0)),
            pl.BlockSpec((1, D_H), lambda i: (0, 0)),
        ],
        out_specs=pl.BlockSpec((RB, D_H), lambda i: (i, 0)),
        out_shape=jax.ShapeDtypeStruct((N, D_H), jnp.float32),
    )(acc2, degt, b.reshape(1, D_H))


def kernel(x, edge_index, W, b):
    ev = edge_index.reshape(2, NCH, 1, CH)
    ev128 = edge_index.reshape(2, NCH // 2, 1, 2 * CH)
    deg2 = _deg_count(ev)
    degt = deg2.reshape(NC, NRB, RB)
    hs2 = _matmul_scaled(x, W, degt)
    acc2 = _edge_accumulate(hs2, ev, ev128)
    return _readout(acc2, degt, b)


# speedup vs baseline: 1.4913x; 1.0244x over previous
import functools

import jax
import jax.numpy as jnp
from jax import lax
from jax.experimental import pallas as pl
from jax.experimental.pallas import tpu as pltpu
from jax.experimental.pallas import tpu_sc as plsc

N = 10000
E = 160000
D_IN = 256
D_H = 256
HALF = 128

NC = 2
NS = 16
L = 16

RPT = 640
RPT_LAST = N - RPT * (NS - 1)
ROWS_PAD = RPT * NS

CH = 64
NCH = E // CH

RB = 1000
NRB = N // RB

CPW1 = NCH // (NC * NS)
PAIRS1 = (CPW1 - 2) // 2

CPT3 = NCH // NS
QUADS3 = CPT3 // 4


def _mesh():
    return plsc.VectorSubcoreMesh(core_axis_name="c", subcore_axis_name="s",
                                  num_cores=NC, num_subcores=NS)


def _row_slab(s):
    return s * RPT, s == NS - 1


def _deg_count(ev):
    @functools.partial(
        pl.kernel,
        out_type=jax.ShapeDtypeStruct((NC * N,), jnp.float32),
        mesh=_mesh(),
        scratch_types=[
            pltpu.VMEM_SHARED((ROWS_PAD,), jnp.float32),
            pltpu.VMEM((RPT,), jnp.float32),
            pltpu.VMEM((CH,), jnp.float32),
            pltpu.VMEM((CPW1 * CH,), jnp.int32),
            pltpu.VMEM((CH,), jnp.int32),
            pltpu.VMEM((CH,), jnp.int32),
            pltpu.SemaphoreType.DMA,
            pltpu.SemaphoreType.DMA,
        ],
    )
    def k(ev_hbm, deg2_hbm, deg, zerov, onesb, dstall, dstb0, dstb1,
          sems0, sems1):
        c = lax.axis_index("c")
        s = lax.axis_index("s")
        base_r, is_last = _row_slab(s)
        w = c * NS + s
        first = w * CPW1

        pltpu.sync_copy(ev_hbm.at[pl.ds(E + first * CH, CPW1 * CH)], dstall)
        for g in range(CH // L):
            onesb[pl.ds(g * L, L)] = jnp.full((L,), 1.0, jnp.float32)
        for g in range(RPT // L):
            zerov[pl.ds(g * L, L)] = jnp.zeros((L,), jnp.float32)
        pltpu.sync_copy(zerov, deg.at[pl.ds(base_r, RPT)])
        plsc.subcore_barrier()

        def move(i, dstb):
            for g in range(CH // L):
                dstb[pl.ds(g * L, L)] = dstall[pl.ds(i * CH + g * L, L)]

        def fire(dstb, sem):
            pltpu.async_copy(onesb, deg.at[dstb], sem, add=True)

        def drain(dstb, sem):
            pltpu.make_async_copy(onesb, deg.at[dstb], sem).wait()

        move(0, dstb0)
        fire(dstb0, sems0)

        def pair(p, carry):
            i0 = 2 * p + 1
            move(i0, dstb1)
            fire(dstb1, sems1)
            drain(dstb0, sems0)
            move(i0 + 1, dstb0)
            fire(dstb0, sems0)
            drain(dstb1, sems1)
            return carry

        lax.fori_loop(0, PAIRS1, pair, 0)
        drain(dstb0, sems0)
        move(CPW1 - 1, dstb0)
        pltpu.sync_copy(onesb, deg.at[dstb0], add=True)

        @pl.when(w < 4)
        def _():
            pltpu.sync_copy(
                ev_hbm.at[pl.ds(E + (NC * NS * CPW1 + w) * CH, CH)], dstb0)
            pltpu.sync_copy(onesb, deg.at[dstb0], add=True)

        plsc.subcore_barrier()

        pltpu.sync_copy(deg.at[pl.ds(base_r, RPT)], zerov)

        @pl.when(jnp.logical_not(is_last))
        def _():
            pltpu.sync_copy(zerov,
                            deg2_hbm.at[pl.ds(c * N + base_r, RPT)])

        @pl.when(is_last)
        def _():
            pltpu.sync_copy(zerov.at[pl.ds(0, RPT_LAST)],
                            deg2_hbm.at[pl.ds(c * N + base_r, RPT_LAST)])

    return k(ev)


def _mm_body(x_ref, w_ref, deg_ref, o_ref):
    i = pl.program_id(0)
    d = deg_ref[0, i, :] + deg_ref[1, i, :] + 1.0
    dis = lax.rsqrt(d)
    h = jnp.dot(x_ref[...], w_ref[...], preferred_element_type=jnp.float32)
    h = h * dis[:, None]
    o_ref[0, :, :] = h[:, :HALF]
    o_ref[1, :, :] = h[:, HALF:]


def _matmul_scaled(x, W, degt):
    return pl.pallas_call(
        _mm_body,
        grid=(NRB,),
        in_specs=[
            pl.BlockSpec((RB, D_IN), lambda i: (i, 0)),
            pl.BlockSpec((D_IN, D_H), lambda i: (0, 0)),
            pl.BlockSpec((NC, NRB, RB), lambda i: (0, 0, 0)),
        ],
        out_specs=pl.BlockSpec((NC, RB, HALF), lambda i: (0, i, 0)),
        out_shape=jax.ShapeDtypeStruct((2, N, HALF), jnp.float32),
    )(x, W, degt)


def _edge_accumulate(hs2, ev):
    @functools.partial(
        pl.kernel,
        out_type=jax.ShapeDtypeStruct((NC, N, HALF), jnp.float32),
        mesh=_mesh(),
        scratch_types=[
            pltpu.VMEM_SHARED((N, HALF), jnp.float32),
            pltpu.VMEM((CH, HALF), jnp.float32),
            pltpu.VMEM((CH, HALF), jnp.float32),
            pltpu.VMEM((CH, HALF), jnp.float32),
            pltpu.VMEM((CH, HALF), jnp.float32),
            pltpu.VMEM((CPT3 * CH,), jnp.int32),
            pltpu.VMEM((CH,), jnp.int32),
            pltpu.VMEM((CH,), jnp.int32),
            pltpu.VMEM((CH,), jnp.int32),
            pltpu.VMEM((CH,), jnp.int32),
            pltpu.VMEM((CH,), jnp.int32),
            pltpu.VMEM((CH,), jnp.int32),
            pltpu.VMEM((CH,), jnp.int32),
            pltpu.VMEM((CH,), jnp.int32),
            pltpu.SemaphoreType.DMA,
            pltpu.SemaphoreType.DMA,
            pltpu.SemaphoreType.DMA,
            pltpu.SemaphoreType.DMA,
            pltpu.SemaphoreType.DMA,
            pltpu.SemaphoreType.DMA,
            pltpu.SemaphoreType.DMA,
            pltpu.SemaphoreType.DMA,
            pltpu.SemaphoreType.DMA,
            pltpu.SemaphoreType.DMA,
            pltpu.SemaphoreType.DMA,
            pltpu.SemaphoreType.DMA,
        ],
    )
    def k(hs_hbm, ev_hbm, acc2_hbm, acc,
          rows0, rows1, rows2, rows3, srcall,
          srcb0, srcb1, srcb2, srcb3, dstb0, dstb1, dstb2, dstb3,
          semg0, semg1, semg2, semg3, semi0, semi1, semi2, semi3,
          sems0, sems1, sems2, sems3):
        rows = [rows0, rows1, rows2, rows3]
        srcb = [srcb0, srcb1, srcb2, srcb3]
        dstb = [dstb0, dstb1, dstb2, dstb3]
        semg = [semg0, semg1, semg2, semg3]
        semi = [semi0, semi1, semi2, semi3]
        sems = [sems0, sems1, sems2, sems3]

        c = lax.axis_index("c")
        s = lax.axis_index("s")
        base_r, is_last = _row_slab(s)
        first = s * CPT3

        pltpu.sync_copy(ev_hbm.at[pl.ds(first * CH, CPT3 * CH)], srcall)

        @pl.when(jnp.logical_not(is_last))
        def _():
            pltpu.sync_copy(hs_hbm.at[c, pl.ds(base_r, RPT)],
                            acc.at[pl.ds(base_r, RPT)])

        @pl.when(is_last)
        def _():
            pltpu.sync_copy(hs_hbm.at[c, pl.ds(base_r, RPT_LAST)],
                            acc.at[pl.ds(base_r, RPT_LAST)])

        plsc.subcore_barrier()

        def launch(i, a):
            idx = srcall.at[pl.ds(i * CH, CH)]
            pltpu.async_copy(hs_hbm.at[c].at[idx], rows[a], semg[a])
            pltpu.async_copy(ev_hbm.at[pl.ds(E + (first + i) * CH, CH)],
                             dstb[a], semi[a])

        def retire(a):
            pltpu.make_async_copy(hs_hbm.at[c].at[srcb[a]], rows[a],
                                  semg[a]).wait()
            pltpu.make_async_copy(ev_hbm.at[pl.ds(0, CH)], dstb[a],
                                  semi[a]).wait()
            pltpu.async_copy(rows[a], acc.at[dstb[a]], sems[a], add=True)

        def wait_scat(a):
            pltpu.make_async_copy(rows[a], acc.at[dstb[a]], sems[a]).wait()

        launch(0, 0)
        launch(1, 1)
        launch(2, 2)

        def quad(q, carry):
            for a in range(4):
                i = 4 * q + a
                retire(a)
                nxt = (a + 3) % 4

                if a == 0:
                    @pl.when(q > 0)
                    def _():
                        wait_scat(nxt)
                    launch(i + 3, nxt)
                else:
                    wait_scat(nxt)

                    @pl.when(q < QUADS3 - 1)
                    def _():
                        launch(i + 3, nxt)
            return carry

        lax.fori_loop(0, QUADS3, quad, 0)
        wait_scat(3)

        @pl.when(s < 4)
        def _():
            pltpu.sync_copy(ev_hbm.at[pl.ds((NS * CPT3 + s) * CH, CH)], srcb0)
            pltpu.sync_copy(ev_hbm.at[pl.ds(E + (NS * CPT3 + s) * CH, CH)],
                            dstb0)
            pltpu.async_copy(hs_hbm.at[c].at[srcb0], rows0, semg0).wait()
            pltpu.sync_copy(rows0, acc.at[dstb0], add=True)

        plsc.subcore_barrier()

        @pl.when(jnp.logical_not(is_last))
        def _():
            pltpu.sync_copy(acc.at[pl.ds(base_r, RPT)],
                            acc2_hbm.at[c, pl.ds(base_r, RPT)])

        @pl.when(is_last)
        def _():
            pltpu.sync_copy(acc.at[pl.ds(base_r, RPT_LAST)],
                            acc2_hbm.at[c, pl.ds(base_r, RPT_LAST)])

    return k(hs2, ev)


def _ro_body(acc_ref, deg_ref, b_ref, o_ref):
    i = pl.program_id(0)
    d = deg_ref[0, i, :] + deg_ref[1, i, :] + 1.0
    dis = lax.rsqrt(d)
    a = jnp.concatenate([acc_ref[0], acc_ref[1]], axis=1)
    o_ref[...] = jnp.maximum(a * dis[:, None] + b_ref[0, :][None, :], 0.0)


def _readout(acc2, degt, b):
    return pl.pallas_call(
        _ro_body,
        grid=(NRB,),
        in_specs=[
            pl.BlockSpec((NC, RB, HALF), lambda i: (0, i, 0)),
            pl.BlockSpec((NC, NRB, RB), lambda i: (0, 0, 0)),
            pl.BlockSpec((1, D_H), lambda i: (0, 0)),
        ],
        out_specs=pl.BlockSpec((RB, D_H), lambda i: (i, 0)),
        out_shape=jax.ShapeDtypeStruct((N, D_H), jnp.float32),
    )(acc2, degt, b.reshape(1, D_H))


def kernel(x, edge_index, W, b):
    ev = edge_index.reshape(2 * E)
    deg2 = _deg_count(ev)
    degt = deg2.reshape(NC, NRB, RB)
    hs2 = _matmul_scaled(x, W, degt)
    acc2 = _edge_accumulate(hs2, ev)
    return _readout(acc2, degt, b)
